# Initial kernel scaffold; baseline (speedup 1.0000x reference)
import jax
import jax.numpy as jnp
from jax.experimental import pallas as pl


def kernel(node_feature, edge_feature, edge_src, edge_dst, W0, b0, W1, b1, W2, b2, eps):
    raise NotImplementedError("write your pallas kernel here")


# speedup vs baseline: 3.5914x; 3.5914x over previous
import functools
import jax
import jax.numpy as jnp
from jax import lax
from jax.experimental import pallas as pl
from jax.experimental.pallas import tpu as pltpu
from jax.experimental.pallas import tpu_sc as plsc

_N = 10000
_E = 320000
_D = 128
_DE = 16
_U = 128
_EDGE_STEPS = 4
_NODE_STEPS = 4

_NC = 2
_NS = 16
_NW = _NC * _NS
_L = 128
_R = _E // _L
_R_PAD = 2504

_mesh = plsc.VectorSubcoreMesh(
    core_axis_name="c", subcore_axis_name="s", num_cores=_NC, num_subcores=_NS
)


def _edge_worker_rows(wid):
    base = 80 * wid - 8 * jnp.maximum(wid - 24, 0)
    cnt = jnp.where(wid < 24, 80, jnp.where(wid < 31, 72, 76))
    return base, cnt


def _node_subcore_rows(s):
    base = 160 * s - 8 * jnp.maximum(s - 8, 0)
    cnt = jnp.where(s < 8, 160, jnp.where(s < 15, 152, 156))
    return base, cnt


def _acc_region(s):
    base = 624 * s + 8 * jnp.minimum(s, 2)
    return base


@functools.partial(
    pl.kernel,
    out_type=jax.ShapeDtypeStruct((_E, _U), jnp.float32),
    mesh=_mesh,
    scratch_types=[
        pltpu.VMEM((80, _L), jnp.int32),
        pltpu.VMEM((2, _L, _U), jnp.float32),
        pltpu.SemaphoreType.DMA,
        pltpu.SemaphoreType.DMA,
    ],
)
def _sc_gather(table, idx_rows, out, idx_v, rows_v, sem0, sem1):
    c = lax.axis_index("c")
    s = lax.axis_index("s")
    wid = c * _NS + s
    base, cnt = _edge_worker_rows(wid)
    pltpu.sync_copy(idx_rows.at[pl.ds(base, 80)], idx_v)
    pltpu.async_copy(table.at[idx_v.at[0]], rows_v.at[0], sem0)

    def body(k, carry):
        j0 = 2 * k
        j1 = j0 + 1

        @pl.when(j1 < cnt)
        def _():
            pltpu.async_copy(table.at[idx_v.at[j1]], rows_v.at[1], sem1)

        pltpu.make_async_copy(table.at[idx_v.at[j0]], rows_v.at[0], sem0).wait()
        pltpu.sync_copy(rows_v.at[0], out.at[pl.ds((base + j0) * _L, _L)])

        @pl.when(j0 + 2 < cnt)
        def _():
            pltpu.async_copy(table.at[idx_v.at[j0 + 2]], rows_v.at[0], sem0)

        @pl.when(j1 < cnt)
        def _():
            pltpu.make_async_copy(
                table.at[idx_v.at[j1]], rows_v.at[1], sem1
            ).wait()
            pltpu.sync_copy(rows_v.at[1], out.at[pl.ds((base + j1) * _L, _L)])

        return carry

    lax.fori_loop(0, (cnt + 1) // 2, body, 0)


@functools.partial(
    pl.kernel,
    out_type=jax.ShapeDtypeStruct((2 * _N, _U), jnp.float32),
    mesh=_mesh,
    scratch_types=[
        pltpu.VMEM((80, _L), jnp.int32),
        pltpu.VMEM((2 * _L, _U), jnp.float32),
        pltpu.VMEM_SHARED((_N, _U), jnp.float32),
        pltpu.SemaphoreType.DMA,
        pltpu.SemaphoreType.DMA,
    ],
)
def _sc_segsum(x, dst_rows, out, idx_v, rows_v, acc, sem0, sem1):
    c = lax.axis_index("c")
    s = lax.axis_index("s")
    wid = c * _NS + s
    base, cnt = _edge_worker_rows(wid)
    nbase = _acc_region(s)
    slot0 = rows_v.at[pl.ds(0, _L)]
    slot1 = rows_v.at[pl.ds(_L, _L)]

    def zrow(i, carry):
        for jj in range(8):
            rows_v[i, pl.ds(jj * 16, 16)] = jnp.zeros((16,), jnp.float32)
        return carry

    lax.fori_loop(0, 2 * _L, zrow, 0)
    for r in range(2):
        pltpu.sync_copy(rows_v, acc.at[pl.ds(nbase + r * 2 * _L, 2 * _L)])

    @pl.when(s < 2)
    def _():
        pltpu.sync_copy(rows_v.at[pl.ds(0, 120)], acc.at[pl.ds(nbase + 512, 120)])

    @pl.when(s >= 2)
    def _():
        pltpu.sync_copy(rows_v.at[pl.ds(0, 112)], acc.at[pl.ds(nbase + 512, 112)])

    pltpu.sync_copy(dst_rows.at[pl.ds(base, 80)], idx_v)
    plsc.subcore_barrier()

    pltpu.async_copy(x.at[pl.ds(base * _L, _L)], slot0, sem0)

    def body(k, carry):
        j0 = 2 * k
        j1 = j0 + 1

        @pl.when(j1 < cnt)
        def _():
            pltpu.async_copy(x.at[pl.ds((base + j1) * _L, _L)], slot1, sem1)

        pltpu.make_async_copy(
            x.at[pl.ds((base + j0) * _L, _L)], slot0, sem0
        ).wait()
        pltpu.sync_copy(slot0, acc.at[idx_v.at[j0]], add=True)

        @pl.when(j0 + 2 < cnt)
        def _():
            pltpu.async_copy(x.at[pl.ds((base + j0 + 2) * _L, _L)], slot0, sem0)

        @pl.when(j1 < cnt)
        def _():
            pltpu.make_async_copy(
                x.at[pl.ds((base + j1) * _L, _L)], slot1, sem1
            ).wait()
            pltpu.sync_copy(slot1, acc.at[idx_v.at[j1]], add=True)

        return carry

    lax.fori_loop(0, (cnt + 1) // 2, body, 0)
    plsc.subcore_barrier()

    @pl.when(s < 2)
    def _():
        pltpu.sync_copy(
            acc.at[pl.ds(nbase, 632)], out.at[pl.ds(c * _N + nbase, 632)]
        )

    @pl.when(s >= 2)
    def _():
        pltpu.sync_copy(
            acc.at[pl.ds(nbase, 624)], out.at[pl.ds(c * _N + nbase, 624)]
        )


@functools.partial(
    pl.kernel,
    out_type=jax.ShapeDtypeStruct((2 * _N, _U), jnp.float32),
    mesh=_mesh,
    scratch_types=[
        pltpu.VMEM((32, _L), jnp.int32),
        pltpu.VMEM((32, _L), jnp.int32),
        pltpu.VMEM((2 * _L, _U), jnp.float32),
        pltpu.VMEM_SHARED((_N, _U), jnp.float32),
        pltpu.SemaphoreType.DMA,
        pltpu.SemaphoreType.DMA,
    ],
)
def _sc_node_step(nf2, init2, src2_rows, dst_rows, out, sidx_v, didx_v, rows_v,
                  acc, sem0, sem1):
    c = lax.axis_index("c")
    s = lax.axis_index("s")
    base, cnt = _node_subcore_rows(s)
    nbase = _acc_region(s)
    slot0 = rows_v.at[pl.ds(0, _L)]
    slot1 = rows_v.at[pl.ds(_L, _L)]

    @pl.when(s < 2)
    def _():
        pltpu.sync_copy(
            init2.at[pl.ds(c * _N + nbase, 632)], acc.at[pl.ds(nbase, 632)]
        )

    @pl.when(s >= 2)
    def _():
        pltpu.sync_copy(
            init2.at[pl.ds(c * _N + nbase, 624)], acc.at[pl.ds(nbase, 624)]
        )

    plsc.subcore_barrier()

    for ck in range(5):
        @pl.when(32 * ck < cnt)
        def _(ck=ck):
            cc = jnp.minimum(cnt - 32 * ck, 32)
            cb = base + 32 * ck
            pltpu.sync_copy(
                src2_rows.at[pl.ds(c * _R_PAD + cb, 32)], sidx_v
            )
            pltpu.sync_copy(dst_rows.at[pl.ds(cb, 32)], didx_v)
            pltpu.async_copy(nf2.at[sidx_v.at[0]], slot0, sem0)

            def body(k, carry):
                j0 = 2 * k
                j1 = j0 + 1

                @pl.when(j1 < cc)
                def _():
                    pltpu.async_copy(nf2.at[sidx_v.at[j1]], slot1, sem1)

                pltpu.make_async_copy(nf2.at[sidx_v.at[j0]], slot0, sem0).wait()
                pltpu.sync_copy(slot0, acc.at[didx_v.at[j0]], add=True)

                @pl.when(j0 + 2 < cc)
                def _():
                    pltpu.async_copy(nf2.at[sidx_v.at[j0 + 2]], slot0, sem0)

                @pl.when(j1 < cc)
                def _():
                    pltpu.make_async_copy(
                        nf2.at[sidx_v.at[j1]], slot1, sem1
                    ).wait()
                    pltpu.sync_copy(slot1, acc.at[didx_v.at[j1]], add=True)

                return carry

            lax.fori_loop(0, (cc + 1) // 2, body, 0)

    plsc.subcore_barrier()

    @pl.when(s < 2)
    def _():
        pltpu.sync_copy(
            acc.at[pl.ds(nbase, 632)], out.at[pl.ds(c * _N + nbase, 632)]
        )

    @pl.when(s >= 2)
    def _():
        pltpu.sync_copy(
            acc.at[pl.ds(nbase, 624)], out.at[pl.ds(c * _N + nbase, 624)]
        )


_BN = 1000
_BE = 2000
_BE2 = 640


def _tc_mm_body(x_ref, w_ref, o_ref):
    o_ref[...] = jnp.dot(x_ref[...], w_ref[...], preferred_element_type=jnp.float32)


def _tc_node_mm(x, w):
    return pl.pallas_call(
        _tc_mm_body,
        out_shape=jax.ShapeDtypeStruct((_N, w.shape[1]), jnp.float32),
        grid=(_N // _BN,),
        in_specs=[
            pl.BlockSpec((_BN, x.shape[1]), lambda i: (i, 0)),
            pl.BlockSpec(w.shape, lambda i: (0, 0)),
        ],
        out_specs=pl.BlockSpec((_BN, w.shape[1]), lambda i: (i, 0)),
    )(x, w)


def _tc_agg_mm_body(p0_ref, p1_ref, w_ref, b_ref, o_ref):
    agg = p0_ref[...] + p1_ref[...]
    o_ref[...] = (
        jnp.dot(agg, w_ref[...], preferred_element_type=jnp.float32) + b_ref[...]
    )


def _tc_agg_mm(p, w, b):
    return pl.pallas_call(
        _tc_agg_mm_body,
        out_shape=jax.ShapeDtypeStruct((_N, _U), jnp.float32),
        grid=(_N // _BN,),
        in_specs=[
            pl.BlockSpec((_BN, _U), lambda i: (i, 0)),
            pl.BlockSpec((_BN, _U), lambda i: (i, 0)),
            pl.BlockSpec((_U, _U), lambda i: (0, 0)),
            pl.BlockSpec((1, _U), lambda i: (0, 0)),
        ],
        out_specs=pl.BlockSpec((_BN, _U), lambda i: (i, 0)),
    )(p[:_N], p[_N:], w, b)


def _tc_h0_body(g_ref, ef_ref, w_ref, b_ref, o_ref):
    t = jnp.dot(ef_ref[...], w_ref[...], preferred_element_type=jnp.float32)
    o_ref[...] = jnp.maximum(g_ref[...] + t + b_ref[...], 0.0)


def _tc_h0(g0, ef, w0b, b0):
    return pl.pallas_call(
        _tc_h0_body,
        out_shape=jax.ShapeDtypeStruct((_E, _U), jnp.float32),
        grid=(_E // _BE,),
        in_specs=[
            pl.BlockSpec((_BE, _U), lambda i: (i, 0)),
            pl.BlockSpec((_BE, _DE), lambda i: (i, 0)),
            pl.BlockSpec((_DE, _U), lambda i: (0, 0)),
            pl.BlockSpec((1, _U), lambda i: (0, 0)),
        ],
        out_specs=pl.BlockSpec((_BE, _U), lambda i: (i, 0)),
    )(g0, ef, w0b, b0)


def _tc_edge_update_body(h_ref, g_ref, w_ref, o_ref):
    h = h_ref[...]
    hlo = h[:, :_U]
    hhi = h[:, _U:]
    wlo = jnp.dot(hlo, w_ref[...], preferred_element_type=jnp.float32)
    whi = jnp.dot(hhi, w_ref[...], preferred_element_type=jnp.float32)
    hw_rev = jnp.concatenate([whi, wlo], axis=1)
    o_ref[...] = jnp.maximum(g_ref[...] - hw_rev + h, 0.0)


def _tc_edge_update(h2, g2, w1):
    e2 = _E // 2
    return pl.pallas_call(
        _tc_edge_update_body,
        out_shape=jax.ShapeDtypeStruct((e2, 2 * _U), jnp.float32),
        grid=(e2 // _BE2,),
        in_specs=[
            pl.BlockSpec((_BE2, 2 * _U), lambda i: (i, 0)),
            pl.BlockSpec((_BE2, 2 * _U), lambda i: (i, 0)),
            pl.BlockSpec((_U, _U), lambda i: (0, 0)),
        ],
        out_specs=pl.BlockSpec((_BE2, 2 * _U), lambda i: (i, 0)),
    )(h2, g2, w1)


def _tc_nfinit_body(nd_ref, p0_ref, p1_ref, waa_ref, wba_ref, wab_ref, wbb_ref,
                    b2a_ref, b2b_ref, ia_ref, ib_ref, m_ref):
    m = p0_ref[...] + p1_ref[...]
    nd = nd_ref[...]
    m_ref[...] = m
    ia_ref[...] = (
        jnp.dot(nd, waa_ref[...], preferred_element_type=jnp.float32)
        + jnp.dot(m, wba_ref[...], preferred_element_type=jnp.float32)
        + b2a_ref[...]
    )
    ib_ref[...] = (
        jnp.dot(nd, wab_ref[...], preferred_element_type=jnp.float32)
        + jnp.dot(m, wbb_ref[...], preferred_element_type=jnp.float32)
        + b2b_ref[...]
    )


def _tc_nfinit(node, p, w2s, b2a, b2b):
    waa, wba, wab, wbb = w2s
    full = lambda shp: pl.BlockSpec(shp, lambda i: (0, 0))
    out_shape = [
        jax.ShapeDtypeStruct((_N, _U), jnp.float32),
        jax.ShapeDtypeStruct((_N, _U), jnp.float32),
        jax.ShapeDtypeStruct((_N, _U), jnp.float32),
    ]
    blk = pl.BlockSpec((_BN, _U), lambda i: (i, 0))
    return pl.pallas_call(
        _tc_nfinit_body,
        out_shape=out_shape,
        grid=(_N // _BN,),
        in_specs=[blk, blk, blk, full((_U, _U)), full((_U, _U)),
                  full((_U, _U)), full((_U, _U)), full((1, _U)), full((1, _U))],
        out_specs=[blk, blk, blk],
    )(node, p[:_N], p[_N:], waa, wba, wab, wbb, b2a, b2b)


def kernel(node_feature, edge_feature, edge_src, edge_dst, W0, b0, W1, b1, W2,
           b2, eps):
    src = edge_src.astype(jnp.int32)
    dst = edge_dst.astype(jnp.int32)
    pad = (_R_PAD - _R) * _L
    src_p = jnp.concatenate([src, jnp.zeros((pad,), jnp.int32)])
    dst_p = jnp.concatenate([dst, jnp.zeros((pad,), jnp.int32)])
    src_rows = src_p.reshape(_R_PAD, _L)
    dst_rows = dst_p.reshape(_R_PAD, _L)
    src2_rows = jnp.concatenate([src_p, src_p + _N]).reshape(2 * _R_PAD, _L)

    w0a = W0[:_D]
    w0b = W0[_D:]
    b0r = b0.reshape(1, _U)
    b1r = b1.reshape(1, _U)
    scale = 1.0 + eps
    w2 = W2 * scale
    w2s = (w2[:_D, :_U], w2[_D:, :_U], w2[:_D, _U:], w2[_D:, _U:])
    b2a = b2[:_U].reshape(1, _U)
    b2b = b2[_U:].reshape(1, _U)

    proj = _tc_node_mm(node_feature, w0a)
    g0 = _sc_gather(proj, src_rows)
    h = _tc_h0(g0, edge_feature, w0b, b0r)

    for _ in range(_EDGE_STEPS):
        p = _sc_segsum(h, dst_rows)
        bt = _tc_agg_mm(p, W1, b1r)
        g = _sc_gather(bt, src_rows)
        h2 = _tc_edge_update(
            h.reshape(_E // 2, 2 * _U), g.reshape(_E // 2, 2 * _U), W1
        )
        h = h2.reshape(_E, _U)

    p = _sc_segsum(h, dst_rows)
    ia, ib, msum = _tc_nfinit(node_feature, p, w2s, b2a, b2b)
    nf2 = jnp.concatenate([node_feature, msum], axis=0)
    init2 = jnp.concatenate([ia, ib], axis=0)

    for _ in range(_NODE_STEPS):
        nf2 = _sc_node_step(nf2, init2, src2_rows, dst_rows)

    return jnp.concatenate([nf2[:_N], nf2[_N:]], axis=1)


# speedup vs baseline: 4.1885x; 1.1663x over previous
import functools
import jax
import jax.numpy as jnp
from jax import lax
from jax.experimental import pallas as pl
from jax.experimental.pallas import tpu as pltpu
from jax.experimental.pallas import tpu_sc as plsc

_N = 10000
_E = 320000
_D = 128
_DE = 16
_U = 128
_EDGE_STEPS = 4
_NODE_STEPS = 4

_NC = 2
_NS = 16
_NW = _NC * _NS
_L = 128
_R = _E // _L
_R_PAD = 2504

_mesh = plsc.VectorSubcoreMesh(
    core_axis_name="c", subcore_axis_name="s", num_cores=_NC, num_subcores=_NS
)


def _edge_worker_rows(wid):
    base = 80 * wid - 8 * jnp.maximum(wid - 24, 0)
    cnt = jnp.where(wid < 24, 80, jnp.where(wid < 31, 72, 76))
    return base, cnt


def _node_subcore_rows(s):
    base = 160 * s - 8 * jnp.maximum(s - 8, 0)
    cnt = jnp.where(s < 8, 160, jnp.where(s < 15, 152, 156))
    return base, cnt


def _acc_region(s):
    base = 624 * s + 8 * jnp.minimum(s, 2)
    return base


@functools.partial(
    pl.kernel,
    out_type=jax.ShapeDtypeStruct((_E, _U), jnp.float32),
    mesh=_mesh,
    scratch_types=[
        pltpu.VMEM((80, _L), jnp.int32),
        pltpu.VMEM((2, _L, _U), jnp.float32),
        pltpu.VMEM_SHARED((_N, _U), jnp.float32),
        pltpu.SemaphoreType.DMA,
        pltpu.SemaphoreType.DMA,
    ],
)
def _sc_gather(table_hbm, idx_rows, out, idx_v, rows_v, table, sem0, sem1):
    c = lax.axis_index("c")
    s = lax.axis_index("s")
    wid = c * _NS + s
    base, cnt = _edge_worker_rows(wid)
    nbase = _acc_region(s)

    @pl.when(s < 2)
    def _():
        pltpu.sync_copy(table_hbm.at[pl.ds(nbase, 632)], table.at[pl.ds(nbase, 632)])

    @pl.when(s >= 2)
    def _():
        pltpu.sync_copy(table_hbm.at[pl.ds(nbase, 624)], table.at[pl.ds(nbase, 624)])

    pltpu.sync_copy(idx_rows.at[pl.ds(base, 80)], idx_v)
    plsc.subcore_barrier()
    pltpu.async_copy(table.at[idx_v.at[0]], rows_v.at[0], sem0)

    def body(k, carry):
        j0 = 2 * k
        j1 = j0 + 1

        @pl.when(j1 < cnt)
        def _():
            pltpu.async_copy(table.at[idx_v.at[j1]], rows_v.at[1], sem1)

        pltpu.make_async_copy(table.at[idx_v.at[j0]], rows_v.at[0], sem0).wait()
        pltpu.sync_copy(rows_v.at[0], out.at[pl.ds((base + j0) * _L, _L)])

        @pl.when(j0 + 2 < cnt)
        def _():
            pltpu.async_copy(table.at[idx_v.at[j0 + 2]], rows_v.at[0], sem0)

        @pl.when(j1 < cnt)
        def _():
            pltpu.make_async_copy(
                table.at[idx_v.at[j1]], rows_v.at[1], sem1
            ).wait()
            pltpu.sync_copy(rows_v.at[1], out.at[pl.ds((base + j1) * _L, _L)])

        return carry

    lax.fori_loop(0, (cnt + 1) // 2, body, 0)


@functools.partial(
    pl.kernel,
    out_type=jax.ShapeDtypeStruct((2 * _N, _U), jnp.float32),
    mesh=_mesh,
    scratch_types=[
        pltpu.VMEM((80, _L), jnp.int32),
        pltpu.VMEM((2 * _L, _U), jnp.float32),
        pltpu.VMEM_SHARED((_N, _U), jnp.float32),
        pltpu.SemaphoreType.DMA,
        pltpu.SemaphoreType.DMA,
    ],
)
def _sc_segsum(x, dst_rows, out, idx_v, rows_v, acc, sem0, sem1):
    c = lax.axis_index("c")
    s = lax.axis_index("s")
    wid = c * _NS + s
    base, cnt = _edge_worker_rows(wid)
    nbase = _acc_region(s)
    slot0 = rows_v.at[pl.ds(0, _L)]
    slot1 = rows_v.at[pl.ds(_L, _L)]

    def zrow(i, carry):
        for jj in range(8):
            rows_v[i, pl.ds(jj * 16, 16)] = jnp.zeros((16,), jnp.float32)
        return carry

    lax.fori_loop(0, 2 * _L, zrow, 0)
    for r in range(2):
        pltpu.sync_copy(rows_v, acc.at[pl.ds(nbase + r * 2 * _L, 2 * _L)])

    @pl.when(s < 2)
    def _():
        pltpu.sync_copy(rows_v.at[pl.ds(0, 120)], acc.at[pl.ds(nbase + 512, 120)])

    @pl.when(s >= 2)
    def _():
        pltpu.sync_copy(rows_v.at[pl.ds(0, 112)], acc.at[pl.ds(nbase + 512, 112)])

    pltpu.sync_copy(dst_rows.at[pl.ds(base, 80)], idx_v)
    plsc.subcore_barrier()

    pltpu.async_copy(x.at[pl.ds(base * _L, _L)], slot0, sem0)

    def body(k, carry):
        j0 = 2 * k
        j1 = j0 + 1

        @pl.when(j1 < cnt)
        def _():
            pltpu.async_copy(x.at[pl.ds((base + j1) * _L, _L)], slot1, sem1)

        pltpu.make_async_copy(
            x.at[pl.ds((base + j0) * _L, _L)], slot0, sem0
        ).wait()
        pltpu.sync_copy(slot0, acc.at[idx_v.at[j0]], add=True)

        @pl.when(j0 + 2 < cnt)
        def _():
            pltpu.async_copy(x.at[pl.ds((base + j0 + 2) * _L, _L)], slot0, sem0)

        @pl.when(j1 < cnt)
        def _():
            pltpu.make_async_copy(
                x.at[pl.ds((base + j1) * _L, _L)], slot1, sem1
            ).wait()
            pltpu.sync_copy(slot1, acc.at[idx_v.at[j1]], add=True)

        return carry

    lax.fori_loop(0, (cnt + 1) // 2, body, 0)
    plsc.subcore_barrier()

    @pl.when(s < 2)
    def _():
        pltpu.sync_copy(
            acc.at[pl.ds(nbase, 632)], out.at[pl.ds(c * _N + nbase, 632)]
        )

    @pl.when(s >= 2)
    def _():
        pltpu.sync_copy(
            acc.at[pl.ds(nbase, 624)], out.at[pl.ds(c * _N + nbase, 624)]
        )


@functools.partial(
    pl.kernel,
    out_type=jax.ShapeDtypeStruct((2 * _N, _U), jnp.float32),
    mesh=_mesh,
    scratch_types=[
        pltpu.VMEM((32, _L), jnp.int32),
        pltpu.VMEM((32, _L), jnp.int32),
        pltpu.VMEM((2 * _L, _U), jnp.float32),
        pltpu.VMEM_SHARED((_N, _U), jnp.float32),
        pltpu.SemaphoreType.DMA,
        pltpu.SemaphoreType.DMA,
    ],
)
def _sc_node_steps(nf2, init2, src2_rows, dst_rows, out, sidx_v, didx_v, rows_v,
                   acc, sem0, sem1):
    c = lax.axis_index("c")
    s = lax.axis_index("s")
    base, cnt = _node_subcore_rows(s)
    nbase = _acc_region(s)
    slot0 = rows_v.at[pl.ds(0, _L)]
    slot1 = rows_v.at[pl.ds(_L, _L)]

    for t in range(_NODE_STEPS):
        table = nf2 if t == 0 else out

        @pl.when(s < 2)
        def _():
            pltpu.sync_copy(
                init2.at[pl.ds(c * _N + nbase, 632)], acc.at[pl.ds(nbase, 632)]
            )

        @pl.when(s >= 2)
        def _():
            pltpu.sync_copy(
                init2.at[pl.ds(c * _N + nbase, 624)], acc.at[pl.ds(nbase, 624)]
            )

        plsc.subcore_barrier()

        for ck in range(5):
            @pl.when(32 * ck < cnt)
            def _(ck=ck, table=table):
                cc = jnp.minimum(cnt - 32 * ck, 32)
                cb = base + 32 * ck
                pltpu.sync_copy(
                    src2_rows.at[pl.ds(c * _R_PAD + cb, 32)], sidx_v
                )
                pltpu.sync_copy(dst_rows.at[pl.ds(cb, 32)], didx_v)
                pltpu.async_copy(table.at[sidx_v.at[0]], slot0, sem0)

                def body(k, carry):
                    j0 = 2 * k
                    j1 = j0 + 1

                    @pl.when(j1 < cc)
                    def _():
                        pltpu.async_copy(table.at[sidx_v.at[j1]], slot1, sem1)

                    pltpu.make_async_copy(
                        table.at[sidx_v.at[j0]], slot0, sem0
                    ).wait()
                    pltpu.sync_copy(slot0, acc.at[didx_v.at[j0]], add=True)

                    @pl.when(j0 + 2 < cc)
                    def _():
                        pltpu.async_copy(table.at[sidx_v.at[j0 + 2]], slot0, sem0)

                    @pl.when(j1 < cc)
                    def _():
                        pltpu.make_async_copy(
                            table.at[sidx_v.at[j1]], slot1, sem1
                        ).wait()
                        pltpu.sync_copy(slot1, acc.at[didx_v.at[j1]], add=True)

                    return carry

                lax.fori_loop(0, (cc + 1) // 2, body, 0)

        plsc.subcore_barrier()

        @pl.when(s < 2)
        def _():
            pltpu.sync_copy(
                acc.at[pl.ds(nbase, 632)], out.at[pl.ds(c * _N + nbase, 632)]
            )

        @pl.when(s >= 2)
        def _():
            pltpu.sync_copy(
                acc.at[pl.ds(nbase, 624)], out.at[pl.ds(c * _N + nbase, 624)]
            )

        plsc.subcore_barrier()


_BN = 1000
_BE = 4000
_BE2 = 2000


def _tc_mm_body(x_ref, w_ref, o_ref):
    o_ref[...] = jnp.dot(
        x_ref[...], w_ref[...], preferred_element_type=jnp.float32
    )


def _tc_node_mm(x, w):
    return pl.pallas_call(
        _tc_mm_body,
        out_shape=jax.ShapeDtypeStruct((_N, w.shape[1]), jnp.float32),
        grid=(_N // _BN,),
        in_specs=[
            pl.BlockSpec((_BN, x.shape[1]), lambda i: (i, 0)),
            pl.BlockSpec(w.shape, lambda i: (0, 0)),
        ],
        out_specs=pl.BlockSpec((_BN, w.shape[1]), lambda i: (i, 0)),
    )(x, w)


def _tc_agg_mm_body(p0_ref, p1_ref, w_ref, b_ref, o_ref):
    agg = p0_ref[...] + p1_ref[...]
    o_ref[...] = (
        jnp.dot(agg, w_ref[...], preferred_element_type=jnp.float32) + b_ref[...]
    )


def _tc_agg_mm(p, w, b):
    return pl.pallas_call(
        _tc_agg_mm_body,
        out_shape=jax.ShapeDtypeStruct((_N, _U), jnp.float32),
        grid=(_N // _BN,),
        in_specs=[
            pl.BlockSpec((_BN, _U), lambda i: (i, 0)),
            pl.BlockSpec((_BN, _U), lambda i: (i, 0)),
            pl.BlockSpec((_U, _U), lambda i: (0, 0)),
            pl.BlockSpec((1, _U), lambda i: (0, 0)),
        ],
        out_specs=pl.BlockSpec((_BN, _U), lambda i: (i, 0)),
    )(p[:_N], p[_N:], w, b)


def _tc_h0_body(g_ref, ef_ref, w_ref, b_ref, o_ref):
    t = jnp.dot(ef_ref[...], w_ref[...], preferred_element_type=jnp.float32)
    o_ref[...] = jnp.maximum(g_ref[...] + t + b_ref[...], 0.0)


def _tc_h0(g0, ef, w0b, b0):
    return pl.pallas_call(
        _tc_h0_body,
        out_shape=jax.ShapeDtypeStruct((_E, _U), jnp.float32),
        grid=(_E // _BE,),
        in_specs=[
            pl.BlockSpec((_BE, _U), lambda i: (i, 0)),
            pl.BlockSpec((_BE, _DE), lambda i: (i, 0)),
            pl.BlockSpec((_DE, _U), lambda i: (0, 0)),
            pl.BlockSpec((1, _U), lambda i: (0, 0)),
        ],
        out_specs=pl.BlockSpec((_BE, _U), lambda i: (i, 0)),
    )(g0, ef, w0b, b0)


def _tc_edge_update_body(h_ref, g_ref, w_ref, o_ref):
    h = h_ref[...]
    hw_rev = jnp.dot(h, w_ref[...], preferred_element_type=jnp.float32)
    o_ref[...] = jnp.maximum(g_ref[...] - hw_rev + h, 0.0)


def _tc_edge_update(h2, g2, w1r):
    e2 = _E // 2
    return pl.pallas_call(
        _tc_edge_update_body,
        out_shape=jax.ShapeDtypeStruct((e2, 2 * _U), jnp.float32),
        grid=(e2 // _BE2,),
        in_specs=[
            pl.BlockSpec((_BE2, 2 * _U), lambda i: (i, 0)),
            pl.BlockSpec((_BE2, 2 * _U), lambda i: (i, 0)),
            pl.BlockSpec((2 * _U, 2 * _U), lambda i: (0, 0)),
        ],
        out_specs=pl.BlockSpec((_BE2, 2 * _U), lambda i: (i, 0)),
    )(h2, g2, w1r)


def _tc_nfinit_body(nd_ref, p0_ref, p1_ref, waa_ref, wba_ref, wab_ref, wbb_ref,
                    b2a_ref, b2b_ref, ia_ref, ib_ref, m_ref):
    m = p0_ref[...] + p1_ref[...]
    nd = nd_ref[...]
    m_ref[...] = m
    ia_ref[...] = (
        jnp.dot(nd, waa_ref[...], preferred_element_type=jnp.float32)
        + jnp.dot(m, wba_ref[...], preferred_element_type=jnp.float32)
        + b2a_ref[...]
    )
    ib_ref[...] = (
        jnp.dot(nd, wab_ref[...], preferred_element_type=jnp.float32)
        + jnp.dot(m, wbb_ref[...], preferred_element_type=jnp.float32)
        + b2b_ref[...]
    )


def _tc_nfinit(node, p, w2s, b2a, b2b):
    waa, wba, wab, wbb = w2s
    full = lambda shp: pl.BlockSpec(shp, lambda i: (0, 0))
    out_shape = [
        jax.ShapeDtypeStruct((_N, _U), jnp.float32),
        jax.ShapeDtypeStruct((_N, _U), jnp.float32),
        jax.ShapeDtypeStruct((_N, _U), jnp.float32),
    ]
    blk = pl.BlockSpec((_BN, _U), lambda i: (i, 0))
    return pl.pallas_call(
        _tc_nfinit_body,
        out_shape=out_shape,
        grid=(_N // _BN,),
        in_specs=[blk, blk, blk, full((_U, _U)), full((_U, _U)),
                  full((_U, _U)), full((_U, _U)), full((1, _U)), full((1, _U))],
        out_specs=[blk, blk, blk],
    )(node, p[:_N], p[_N:], waa, wba, wab, wbb, b2a, b2b)


def kernel(node_feature, edge_feature, edge_src, edge_dst, W0, b0, W1, b1, W2,
           b2, eps):
    src = edge_src.astype(jnp.int32)
    dst = edge_dst.astype(jnp.int32)
    pad = (_R_PAD - _R) * _L
    src_p = jnp.concatenate([src, jnp.zeros((pad,), jnp.int32)])
    dst_p = jnp.concatenate([dst, jnp.zeros((pad,), jnp.int32)])
    src_rows = src_p.reshape(_R_PAD, _L)
    dst_rows = dst_p.reshape(_R_PAD, _L)
    src2_rows = jnp.concatenate([src_p, src_p + _N]).reshape(2 * _R_PAD, _L)

    w0a = W0[:_D]
    w0b = W0[_D:]
    b0r = b0.reshape(1, _U)
    b1r = b1.reshape(1, _U)
    zu = jnp.zeros((_U, _U), jnp.float32)
    w1r = jnp.concatenate(
        [jnp.concatenate([zu, W1], axis=1), jnp.concatenate([W1, zu], axis=1)],
        axis=0,
    )
    scale = 1.0 + eps
    w2 = W2 * scale
    w2s = (w2[:_D, :_U], w2[_D:, :_U], w2[:_D, _U:], w2[_D:, _U:])
    b2a = b2[:_U].reshape(1, _U)
    b2b = b2[_U:].reshape(1, _U)

    proj = _tc_node_mm(node_feature, w0a)
    g0 = _sc_gather(proj, src_rows)
    h = _tc_h0(g0, edge_feature, w0b, b0r)

    for _ in range(_EDGE_STEPS):
        p = _sc_segsum(h, dst_rows)
        bt = _tc_agg_mm(p, W1, b1r)
        g = _sc_gather(bt, src_rows)
        h2 = _tc_edge_update(
            h.reshape(_E // 2, 2 * _U), g.reshape(_E // 2, 2 * _U), w1r
        )
        h = h2.reshape(_E, _U)

    p = _sc_segsum(h, dst_rows)
    ia, ib, msum = _tc_nfinit(node_feature, p, w2s, b2a, b2b)
    nf2 = jnp.concatenate([node_feature, msum], axis=0)
    init2 = jnp.concatenate([ia, ib], axis=0)

    nf2 = _sc_node_steps(nf2, init2, src2_rows, dst_rows)

    return jnp.concatenate([nf2[:_N], nf2[_N:]], axis=1)
